# Initial kernel scaffold; baseline (speedup 1.0000x reference)
import jax
import jax.numpy as jnp
from jax.experimental import pallas as pl


def kernel(x, edge_index, Wxi, bxi, Whi, bhi, Wxf, bxf, Whf, bhf, Wxc, bxc, Whc, bhc, Wxo, bxo, Who, bho, wci, wcf, wco, bi, bf, bc, bo):
    raise NotImplementedError("write your pallas kernel here")


# speedup vs baseline: 24.7880x; 24.7880x over previous
import jax
import jax.numpy as jnp
from jax import lax
from jax.experimental import pallas as pl
from jax.experimental.pallas import tpu as pltpu
from jax.experimental.pallas import tpu_sc as plsc

N = 10000
N_PAD = 10240
E = 320000
D_IN = 128
D_OUT = 128
NGATE = 3
NSC = 2
NSUB = 16
ROWS16 = N_PAD // 16
EDGES_PER_WORKER = E // (NSC * NSUB)
EDGES_PER_CORE = E // NSC
EDGES_PER_SUB = EDGES_PER_CORE // NSUB
WIN = 200
NWIN = EDGES_PER_SUB // WIN


def _sc_mesh():
    return plsc.VectorSubcoreMesh(core_axis_name="c", subcore_axis_name="s")


_SC_PARAMS = pltpu.CompilerParams(needs_layout_passes=False,
                                  use_tc_tiling_on_sc=False)


def _degree_partials(dst):
    iota = lax.iota(jnp.int32, ROWS16)

    @pl.kernel(
        out_type=jax.ShapeDtypeStruct((NSC * ROWS16, 16), jnp.float32),
        mesh=_sc_mesh(),
        scratch_types=[
            pltpu.VMEM((EDGES_PER_WORKER,), jnp.int32),
            pltpu.VMEM((ROWS16, 16), jnp.float32),
            pltpu.VMEM((ROWS16,), jnp.int32),
            pltpu.VMEM_SHARED((ROWS16, 16), jnp.float32),
        ],
        compiler_params=_SC_PARAMS,
    )
    def deg_kernel(dst_hbm, iota_hbm, out_hbm, dst_v, hist_v, iota_v, deg_sh):
        cid = lax.axis_index("c")
        sid = lax.axis_index("s")
        wid = cid * NSUB + sid

        zeros16 = jnp.zeros((16,), jnp.float32)

        @pl.loop(0, ROWS16)
        def _(r):
            hist_v[r] = zeros16

        @pl.when(sid == 0)
        def _():
            pltpu.sync_copy(hist_v, deg_sh)

        plsc.subcore_barrier()

        pltpu.sync_copy(dst_hbm.at[pl.ds(wid * EDGES_PER_WORKER, EDGES_PER_WORKER)], dst_v)
        pltpu.sync_copy(iota_hbm, iota_v)

        ones16 = jnp.ones((16,), jnp.float32)

        @pl.loop(0, EDGES_PER_WORKER // 16)
        def _(i):
            v = dst_v[pl.ds(i * 16, 16)]
            row = jnp.right_shift(v, 4)
            lane = jnp.bitwise_and(v, 15)
            plsc.addupdate_scatter(hist_v, [row, lane], ones16)

        pltpu.sync_copy(hist_v, deg_sh.at[iota_v], add=True)
        plsc.subcore_barrier()

        rows_per_sub = ROWS16 // NSUB
        pltpu.sync_copy(
            deg_sh.at[pl.ds(sid * rows_per_sub, rows_per_sub)],
            out_hbm.at[pl.ds(cid * ROWS16 + sid * rows_per_sub, rows_per_sub)],
        )

    return deg_kernel(dst, iota)


def _matmul_scale(x_pad, w_cat, deg_parts):
    BLK = 2048
    grid = (N_PAD // BLK,)

    def body(x_ref, w_ref, d_ref, hi_ref, hc_ref, ho_ref, dv_ref):
        h = jnp.dot(x_ref[...], w_ref[...], preferred_element_type=jnp.float32)
        deg = d_ref[0] + d_ref[1]
        dinv = jnp.where(deg > 0.0, lax.rsqrt(jnp.maximum(deg, 1.0)), 0.0)
        hs = h * dinv
        hi_ref[...] = hs[:, :128]
        hc_ref[...] = hs[:, 128:256]
        ho_ref[...] = hs[:, 256:]
        dv_ref[...] = jnp.broadcast_to(dinv, (BLK, 128))

    out128 = jax.ShapeDtypeStruct((N_PAD, 128), jnp.float32)
    blk128 = pl.BlockSpec((BLK, 128), lambda i: (i, 0))
    return pl.pallas_call(
        body,
        grid=grid,
        in_specs=[
            pl.BlockSpec((BLK, D_IN), lambda i: (i, 0)),
            pl.BlockSpec((D_IN, NGATE * D_OUT), lambda i: (0, 0)),
            pl.BlockSpec((NSC, BLK, 1), lambda i: (0, i, 0)),
        ],
        out_specs=[blk128, blk128, blk128, blk128],
        out_shape=[out128, out128, out128, out128],
    )(x_pad, w_cat, deg_parts)


def _aggregate(h_i, h_c, h_o, src, dst, zeros_init):
    rows_per_sub = N_PAD // NSUB

    @pl.kernel(
        out_type=jax.ShapeDtypeStruct((NGATE * NSC * N_PAD, 128), jnp.float32),
        mesh=_sc_mesh(),
        scratch_types=[
            pltpu.VMEM((WIN,), jnp.int32),
            pltpu.VMEM((WIN,), jnp.int32),
            pltpu.VMEM((WIN, 128), jnp.float32),
            pltpu.VMEM_SHARED((N_PAD, 128), jnp.float32),
        ],
        compiler_params=_SC_PARAMS,
    )
    def agg_kernel(hi_hbm, hc_hbm, ho_hbm, src_hbm, dst_hbm, z_hbm, out_hbm,
                   src_v, dst_v, rows_v, acc_sh):
        cid = lax.axis_index("c")
        sid = lax.axis_index("s")
        slc = pl.ds(sid * rows_per_sub, rows_per_sub)

        for g, h_hbm in enumerate((hi_hbm, hc_hbm, ho_hbm)):
            pltpu.sync_copy(z_hbm, acc_sh.at[slc])
            plsc.subcore_barrier()

            @pl.loop(0, NWIN)
            def _(w):
                base = cid * EDGES_PER_CORE + sid * EDGES_PER_SUB + w * WIN
                pltpu.sync_copy(src_hbm.at[pl.ds(base, WIN)], src_v)
                pltpu.sync_copy(dst_hbm.at[pl.ds(base, WIN)], dst_v)
                pltpu.sync_copy(h_hbm.at[src_v], rows_v)
                pltpu.sync_copy(rows_v, acc_sh.at[dst_v], add=True)

            plsc.subcore_barrier()
            out_base = (g * NSC + cid) * N_PAD + sid * rows_per_sub
            pltpu.sync_copy(acc_sh.at[slc],
                            out_hbm.at[pl.ds(out_base, rows_per_sub)])
            plsc.subcore_barrier()

    return agg_kernel(h_i, h_c, h_o, src, dst, zeros_init)


def _gates(parts, dinv_b, b_i, b_c, b_o, wco):
    BLK = 2048
    grid = (N_PAD // BLK,)

    def body(p_ref, dv_ref, bi_ref, bc_ref, bo_ref, wco_ref, h_ref, c_ref):
        dv = dv_ref[...]
        acc_i = p_ref[0, 0] + p_ref[0, 1]
        acc_c = p_ref[1, 0] + p_ref[1, 1]
        acc_o = p_ref[2, 0] + p_ref[2, 1]
        i_gate = jax.nn.sigmoid(acc_i * dv + bi_ref[...])
        t_gate = jnp.tanh(acc_c * dv + bc_ref[...])
        c_new = i_gate * t_gate
        o_gate = jax.nn.sigmoid(acc_o * dv + wco_ref[...] * c_new + bo_ref[...])
        h_ref[...] = o_gate * jnp.tanh(c_new)
        c_ref[...] = c_new

    bias_spec = pl.BlockSpec((1, 128), lambda i: (0, 0))
    blk128 = pl.BlockSpec((BLK, 128), lambda i: (i, 0))
    return pl.pallas_call(
        body,
        grid=grid,
        in_specs=[
            pl.BlockSpec((NGATE, NSC, BLK, 128), lambda i: (0, 0, i, 0)),
            blk128,
            bias_spec, bias_spec, bias_spec, bias_spec,
        ],
        out_specs=[blk128, blk128],
        out_shape=[
            jax.ShapeDtypeStruct((N_PAD, 128), jnp.float32),
            jax.ShapeDtypeStruct((N_PAD, 128), jnp.float32),
        ],
    )(parts, dinv_b, b_i, b_c, b_o, wco)


def kernel(x, edge_index, Wxi, bxi, Whi, bhi, Wxf, bxf, Whf, bhf, Wxc, bxc,
           Whc, bhc, Wxo, bxo, Who, bho, wci, wcf, wco, bi, bf, bc, bo):
    src = edge_index[0].astype(jnp.int32)
    dst = edge_index[1].astype(jnp.int32)

    deg_parts = _degree_partials(dst)
    deg_parts = deg_parts.reshape(NSC, N_PAD, 1)

    x_pad = jnp.pad(x, ((0, N_PAD - N), (0, 0)))
    w_cat = jnp.concatenate([Wxi, Wxc, Wxo], axis=1)
    h_i, h_c, h_o, dinv_b = _matmul_scale(x_pad, w_cat, deg_parts)

    zeros_init = jnp.zeros((N_PAD // NSUB, 128), jnp.float32)
    parts = _aggregate(h_i, h_c, h_o, src, dst, zeros_init)
    parts = parts.reshape(NGATE, NSC, N_PAD, 128)

    b_i = (bi + bxi + bhi).reshape(1, 128)
    b_c = (bc + bxc + bhc).reshape(1, 128)
    b_o = (bo + bxo + bho).reshape(1, 128)
    h_out, c_out = _gates(parts, dinv_b, b_i, b_c, b_o, wco.reshape(1, 128))
    return h_out[:N], c_out[:N]


# speedup vs baseline: 30.7566x; 1.2408x over previous
import jax
import jax.numpy as jnp
from jax import lax
from jax.experimental import pallas as pl
from jax.experimental.pallas import tpu as pltpu
from jax.experimental.pallas import tpu_sc as plsc

N = 10000
N_PAD = 10240
E = 320000
D_IN = 128
D_OUT = 128
NGATE = 3
NSC = 2
NSUB = 16
ROWS16 = N_PAD // 16
EDGES_PER_WORKER = E // (NSC * NSUB)
EDGES_PER_CORE = E // NSC
EDGES_PER_SUB = EDGES_PER_CORE // NSUB
WIN = 200
NWIN = EDGES_PER_SUB // WIN


def _sc_mesh():
    return plsc.VectorSubcoreMesh(core_axis_name="c", subcore_axis_name="s")


_SC_PARAMS = pltpu.CompilerParams(needs_layout_passes=False,
                                  use_tc_tiling_on_sc=False)


def _degree_partials(dst):
    iota = lax.iota(jnp.int32, ROWS16)

    @pl.kernel(
        out_type=jax.ShapeDtypeStruct((NSC * ROWS16, 16), jnp.float32),
        mesh=_sc_mesh(),
        scratch_types=[
            pltpu.VMEM((EDGES_PER_WORKER,), jnp.int32),
            pltpu.VMEM((ROWS16, 16), jnp.float32),
            pltpu.VMEM((ROWS16,), jnp.int32),
            pltpu.VMEM_SHARED((ROWS16, 16), jnp.float32),
        ],
        compiler_params=_SC_PARAMS,
    )
    def deg_kernel(dst_hbm, iota_hbm, out_hbm, dst_v, hist_v, iota_v, deg_sh):
        cid = lax.axis_index("c")
        sid = lax.axis_index("s")
        wid = cid * NSUB + sid

        zeros16 = jnp.zeros((16,), jnp.float32)

        @pl.loop(0, ROWS16)
        def _(r):
            hist_v[r] = zeros16

        @pl.when(sid == 0)
        def _():
            pltpu.sync_copy(hist_v, deg_sh)

        plsc.subcore_barrier()

        pltpu.sync_copy(dst_hbm.at[pl.ds(wid * EDGES_PER_WORKER, EDGES_PER_WORKER)], dst_v)
        pltpu.sync_copy(iota_hbm, iota_v)

        ones16 = jnp.ones((16,), jnp.float32)

        @pl.loop(0, EDGES_PER_WORKER // 16)
        def _(i):
            v = dst_v[pl.ds(i * 16, 16)]
            row = jnp.right_shift(v, 4)
            lane = jnp.bitwise_and(v, 15)
            plsc.addupdate_scatter(hist_v, [row, lane], ones16)

        pltpu.sync_copy(hist_v, deg_sh.at[iota_v], add=True)
        plsc.subcore_barrier()

        rows_per_sub = ROWS16 // NSUB
        pltpu.sync_copy(
            deg_sh.at[pl.ds(sid * rows_per_sub, rows_per_sub)],
            out_hbm.at[pl.ds(cid * ROWS16 + sid * rows_per_sub, rows_per_sub)],
        )

    return deg_kernel(dst, iota)


def _matmul_scale(x_pad, w_cat, deg_parts):
    BLK = 2048
    grid = (N_PAD // BLK,)

    def body(x_ref, w_ref, d_ref, hi_ref, hc_ref, ho_ref, dv_ref):
        h = jnp.dot(x_ref[...], w_ref[...], preferred_element_type=jnp.float32)
        deg = d_ref[0] + d_ref[1]
        dinv = jnp.where(deg > 0.0, lax.rsqrt(jnp.maximum(deg, 1.0)), 0.0)
        hs = h * dinv
        hi_ref[...] = hs[:, :128]
        hc_ref[...] = hs[:, 128:256]
        ho_ref[...] = hs[:, 256:]
        dv_ref[...] = jnp.broadcast_to(dinv, (BLK, 128))

    out128 = jax.ShapeDtypeStruct((N_PAD, 128), jnp.float32)
    blk128 = pl.BlockSpec((BLK, 128), lambda i: (i, 0))
    return pl.pallas_call(
        body,
        grid=grid,
        in_specs=[
            pl.BlockSpec((BLK, D_IN), lambda i: (i, 0)),
            pl.BlockSpec((D_IN, NGATE * D_OUT), lambda i: (0, 0)),
            pl.BlockSpec((NSC, BLK, 1), lambda i: (0, i, 0)),
        ],
        out_specs=[blk128, blk128, blk128, blk128],
        out_shape=[out128, out128, out128, out128],
    )(x_pad, w_cat, deg_parts)


def _aggregate(h_i, h_c, h_o, src2d, dst2d, zeros_init):
    rows_per_sub = N_PAD // NSUB

    @pl.kernel(
        out_type=jax.ShapeDtypeStruct((NGATE * NSC * N_PAD, 128), jnp.float32),
        mesh=_sc_mesh(),
        scratch_types=[
            pltpu.VMEM((NWIN, WIN), jnp.int32),
            pltpu.VMEM((NWIN, WIN), jnp.int32),
            pltpu.VMEM((WIN, 128), jnp.float32),
            pltpu.VMEM_SHARED((N_PAD, 128), jnp.float32),
        ],
        compiler_params=_SC_PARAMS,
    )
    def agg_kernel(hi_hbm, hc_hbm, ho_hbm, src_hbm, dst_hbm, z_hbm, out_hbm,
                   src_v, dst_v, rows_v, acc_sh):
        cid = lax.axis_index("c")
        sid = lax.axis_index("s")
        slc = pl.ds(sid * rows_per_sub, rows_per_sub)

        base_row = (cid * EDGES_PER_CORE + sid * EDGES_PER_SUB) // WIN
        pltpu.sync_copy(src_hbm.at[pl.ds(base_row, NWIN)], src_v)
        pltpu.sync_copy(dst_hbm.at[pl.ds(base_row, NWIN)], dst_v)

        for g, h_hbm in enumerate((hi_hbm, hc_hbm, ho_hbm)):
            pltpu.sync_copy(z_hbm, acc_sh.at[slc])
            plsc.subcore_barrier()

            @pl.loop(0, NWIN)
            def _(w):
                pltpu.sync_copy(h_hbm.at[src_v.at[w]], rows_v)
                pltpu.sync_copy(rows_v, acc_sh.at[dst_v.at[w]], add=True)

            plsc.subcore_barrier()
            out_base = (g * NSC + cid) * N_PAD + sid * rows_per_sub
            pltpu.sync_copy(acc_sh.at[slc],
                            out_hbm.at[pl.ds(out_base, rows_per_sub)])
            plsc.subcore_barrier()

    return agg_kernel(h_i, h_c, h_o, src2d, dst2d, zeros_init)


def _gates(parts, dinv_b, b_i, b_c, b_o, wco):
    BLK = 2048
    grid = (N_PAD // BLK,)

    def body(p_ref, dv_ref, bi_ref, bc_ref, bo_ref, wco_ref, h_ref, c_ref):
        dv = dv_ref[...]
        acc_i = p_ref[0, 0] + p_ref[0, 1]
        acc_c = p_ref[1, 0] + p_ref[1, 1]
        acc_o = p_ref[2, 0] + p_ref[2, 1]
        i_gate = jax.nn.sigmoid(acc_i * dv + bi_ref[...])
        t_gate = jnp.tanh(acc_c * dv + bc_ref[...])
        c_new = i_gate * t_gate
        o_gate = jax.nn.sigmoid(acc_o * dv + wco_ref[...] * c_new + bo_ref[...])
        h_ref[...] = o_gate * jnp.tanh(c_new)
        c_ref[...] = c_new

    bias_spec = pl.BlockSpec((1, 128), lambda i: (0, 0))
    blk128 = pl.BlockSpec((BLK, 128), lambda i: (i, 0))
    return pl.pallas_call(
        body,
        grid=grid,
        in_specs=[
            pl.BlockSpec((NGATE, NSC, BLK, 128), lambda i: (0, 0, i, 0)),
            blk128,
            bias_spec, bias_spec, bias_spec, bias_spec,
        ],
        out_specs=[blk128, blk128],
        out_shape=[
            jax.ShapeDtypeStruct((N_PAD, 128), jnp.float32),
            jax.ShapeDtypeStruct((N_PAD, 128), jnp.float32),
        ],
    )(parts, dinv_b, b_i, b_c, b_o, wco)


def kernel(x, edge_index, Wxi, bxi, Whi, bhi, Wxf, bxf, Whf, bhf, Wxc, bxc,
           Whc, bhc, Wxo, bxo, Who, bho, wci, wcf, wco, bi, bf, bc, bo):
    src = edge_index[0].astype(jnp.int32)
    dst = edge_index[1].astype(jnp.int32)

    deg_parts = _degree_partials(dst)
    deg_parts = deg_parts.reshape(NSC, N_PAD, 1)

    x_pad = jnp.pad(x, ((0, N_PAD - N), (0, 0)))
    w_cat = jnp.concatenate([Wxi, Wxc, Wxo], axis=1)
    h_i, h_c, h_o, dinv_b = _matmul_scale(x_pad, w_cat, deg_parts)

    zeros_init = jnp.zeros((N_PAD // NSUB, 128), jnp.float32)
    parts = _aggregate(h_i, h_c, h_o, src.reshape(E // WIN, WIN),
                       dst.reshape(E // WIN, WIN), zeros_init)
    parts = parts.reshape(NGATE, NSC, N_PAD, 128)

    b_i = (bi + bxi + bhi).reshape(1, 128)
    b_c = (bc + bxc + bhc).reshape(1, 128)
    b_o = (bo + bxo + bho).reshape(1, 128)
    h_out, c_out = _gates(parts, dinv_b, b_i, b_c, b_o, wco.reshape(1, 128))
    return h_out[:N], c_out[:N]


# speedup vs baseline: 33.0596x; 1.0749x over previous
import jax
import jax.numpy as jnp
from jax import lax
from jax.experimental import pallas as pl
from jax.experimental.pallas import tpu as pltpu
from jax.experimental.pallas import tpu_sc as plsc

N = 10000
N_PAD = 10240
E = 320000
D_IN = 128
D_OUT = 128
NGATE = 3
NSC = 2
NSUB = 16
ROWS16 = N_PAD // 16
EDGES_PER_WORKER = E // (NSC * NSUB)
EDGES_PER_CORE = E // NSC
EDGES_PER_SUB = EDGES_PER_CORE // NSUB
WIN = 100
NWIN = EDGES_PER_SUB // WIN


def _sc_mesh():
    return plsc.VectorSubcoreMesh(core_axis_name="c", subcore_axis_name="s")


_SC_PARAMS = pltpu.CompilerParams(needs_layout_passes=False,
                                  use_tc_tiling_on_sc=False)


def _degree_partials(dst):
    iota = lax.iota(jnp.int32, ROWS16)

    @pl.kernel(
        out_type=jax.ShapeDtypeStruct((NSC * ROWS16, 16), jnp.float32),
        mesh=_sc_mesh(),
        scratch_types=[
            pltpu.VMEM((EDGES_PER_WORKER,), jnp.int32),
            pltpu.VMEM((ROWS16, 16), jnp.float32),
            pltpu.VMEM((ROWS16,), jnp.int32),
            pltpu.VMEM_SHARED((ROWS16, 16), jnp.float32),
        ],
        compiler_params=_SC_PARAMS,
    )
    def deg_kernel(dst_hbm, iota_hbm, out_hbm, dst_v, hist_v, iota_v, deg_sh):
        cid = lax.axis_index("c")
        sid = lax.axis_index("s")
        wid = cid * NSUB + sid

        zeros16 = jnp.zeros((16,), jnp.float32)

        @pl.loop(0, ROWS16)
        def _(r):
            hist_v[r] = zeros16

        @pl.when(sid == 0)
        def _():
            pltpu.sync_copy(hist_v, deg_sh)

        plsc.subcore_barrier()

        pltpu.sync_copy(dst_hbm.at[pl.ds(wid * EDGES_PER_WORKER, EDGES_PER_WORKER)], dst_v)
        pltpu.sync_copy(iota_hbm, iota_v)

        ones16 = jnp.ones((16,), jnp.float32)

        @pl.loop(0, EDGES_PER_WORKER // 16)
        def _(i):
            v = dst_v[pl.ds(i * 16, 16)]
            row = jnp.right_shift(v, 4)
            lane = jnp.bitwise_and(v, 15)
            plsc.addupdate_scatter(hist_v, [row, lane], ones16)

        pltpu.sync_copy(hist_v, deg_sh.at[iota_v], add=True)
        plsc.subcore_barrier()

        rows_per_sub = ROWS16 // NSUB
        pltpu.sync_copy(
            deg_sh.at[pl.ds(sid * rows_per_sub, rows_per_sub)],
            out_hbm.at[pl.ds(cid * ROWS16 + sid * rows_per_sub, rows_per_sub)],
        )

    return deg_kernel(dst, iota)


def _matmul_scale(x_pad, w_cat, deg_parts):
    BLK = 2048
    grid = (N_PAD // BLK,)

    def body(x_ref, w_ref, d_ref, hi_ref, hc_ref, ho_ref, dv_ref):
        h = jnp.dot(x_ref[...], w_ref[...], preferred_element_type=jnp.float32)
        deg = d_ref[0] + d_ref[1]
        dinv = jnp.where(deg > 0.0, lax.rsqrt(jnp.maximum(deg, 1.0)), 0.0)
        hs = h * dinv
        hi_ref[...] = hs[:, :128]
        hc_ref[...] = hs[:, 128:256]
        ho_ref[...] = hs[:, 256:]
        dv_ref[...] = jnp.broadcast_to(dinv, (BLK, 128))

    out128 = jax.ShapeDtypeStruct((N_PAD, 128), jnp.float32)
    blk128 = pl.BlockSpec((BLK, 128), lambda i: (i, 0))
    return pl.pallas_call(
        body,
        grid=grid,
        in_specs=[
            pl.BlockSpec((BLK, D_IN), lambda i: (i, 0)),
            pl.BlockSpec((D_IN, NGATE * D_OUT), lambda i: (0, 0)),
            pl.BlockSpec((NSC, BLK, 1), lambda i: (0, i, 0)),
        ],
        out_specs=[blk128, blk128, blk128, blk128],
        out_shape=[out128, out128, out128, out128],
    )(x_pad, w_cat, deg_parts)


def _aggregate(h_i, h_c, h_o, src2d, dst2d, zeros_init):
    rows_per_sub = N_PAD // NSUB

    @pl.kernel(
        out_type=jax.ShapeDtypeStruct((NGATE * NSC * N_PAD, 128), jnp.float32),
        mesh=_sc_mesh(),
        scratch_types=[
            pltpu.VMEM((NWIN, WIN), jnp.int32),
            pltpu.VMEM((NWIN, WIN), jnp.int32),
            pltpu.VMEM((WIN, 128), jnp.float32),
            pltpu.VMEM((WIN, 128), jnp.float32),
            pltpu.VMEM_SHARED((N_PAD, 128), jnp.float32),
            pltpu.SemaphoreType.DMA,
            pltpu.SemaphoreType.DMA,
            pltpu.SemaphoreType.DMA,
            pltpu.SemaphoreType.DMA,
        ],
        compiler_params=_SC_PARAMS,
    )
    def agg_kernel(hi_hbm, hc_hbm, ho_hbm, src_hbm, dst_hbm, z_hbm, out_hbm,
                   src_v, dst_v, buf0, buf1, acc_sh, gsem0, gsem1, ssem0, ssem1):
        cid = lax.axis_index("c")
        sid = lax.axis_index("s")
        slc = pl.ds(sid * rows_per_sub, rows_per_sub)

        base_row = (cid * EDGES_PER_CORE + sid * EDGES_PER_SUB) // WIN
        pltpu.sync_copy(src_hbm.at[pl.ds(base_row, NWIN)], src_v)
        pltpu.sync_copy(dst_hbm.at[pl.ds(base_row, NWIN)], dst_v)

        for g, h_hbm in enumerate((hi_hbm, hc_hbm, ho_hbm)):
            pltpu.sync_copy(z_hbm, acc_sh.at[slc])
            pltpu.async_copy(h_hbm.at[src_v.at[0]], buf0, gsem0)
            plsc.subcore_barrier()

            @pl.loop(0, NWIN, step=2)
            def _(w):
                pltpu.make_async_copy(h_hbm.at[src_v.at[w]], buf0, gsem0).wait()
                sd0 = pltpu.async_copy(buf0, acc_sh.at[dst_v.at[w]], ssem0,
                                       add=True)
                gd1 = pltpu.async_copy(h_hbm.at[src_v.at[w + 1]], buf1, gsem1)
                gd1.wait()
                sd1 = pltpu.async_copy(buf1, acc_sh.at[dst_v.at[w + 1]], ssem1,
                                       add=True)
                sd0.wait()

                @pl.when(w + 2 < NWIN)
                def _():
                    pltpu.async_copy(h_hbm.at[src_v.at[w + 2]], buf0, gsem0)

                sd1.wait()

            plsc.subcore_barrier()
            out_base = (g * NSC + cid) * N_PAD + sid * rows_per_sub
            pltpu.sync_copy(acc_sh.at[slc],
                            out_hbm.at[pl.ds(out_base, rows_per_sub)])
            plsc.subcore_barrier()

    return agg_kernel(h_i, h_c, h_o, src2d, dst2d, zeros_init)


def _gates(parts, dinv_b, b_i, b_c, b_o, wco):
    BLK = 2048
    grid = (N_PAD // BLK,)

    def body(p_ref, dv_ref, bi_ref, bc_ref, bo_ref, wco_ref, h_ref, c_ref):
        dv = dv_ref[...]
        acc_i = p_ref[0, 0] + p_ref[0, 1]
        acc_c = p_ref[1, 0] + p_ref[1, 1]
        acc_o = p_ref[2, 0] + p_ref[2, 1]
        i_gate = jax.nn.sigmoid(acc_i * dv + bi_ref[...])
        t_gate = jnp.tanh(acc_c * dv + bc_ref[...])
        c_new = i_gate * t_gate
        o_gate = jax.nn.sigmoid(acc_o * dv + wco_ref[...] * c_new + bo_ref[...])
        h_ref[...] = o_gate * jnp.tanh(c_new)
        c_ref[...] = c_new

    bias_spec = pl.BlockSpec((1, 128), lambda i: (0, 0))
    blk128 = pl.BlockSpec((BLK, 128), lambda i: (i, 0))
    return pl.pallas_call(
        body,
        grid=grid,
        in_specs=[
            pl.BlockSpec((NGATE, NSC, BLK, 128), lambda i: (0, 0, i, 0)),
            blk128,
            bias_spec, bias_spec, bias_spec, bias_spec,
        ],
        out_specs=[blk128, blk128],
        out_shape=[
            jax.ShapeDtypeStruct((N_PAD, 128), jnp.float32),
            jax.ShapeDtypeStruct((N_PAD, 128), jnp.float32),
        ],
    )(parts, dinv_b, b_i, b_c, b_o, wco)


def kernel(x, edge_index, Wxi, bxi, Whi, bhi, Wxf, bxf, Whf, bhf, Wxc, bxc,
           Whc, bhc, Wxo, bxo, Who, bho, wci, wcf, wco, bi, bf, bc, bo):
    src = edge_index[0].astype(jnp.int32)
    dst = edge_index[1].astype(jnp.int32)

    deg_parts = _degree_partials(dst)
    deg_parts = deg_parts.reshape(NSC, N_PAD, 1)

    x_pad = jnp.pad(x, ((0, N_PAD - N), (0, 0)))
    w_cat = jnp.concatenate([Wxi, Wxc, Wxo], axis=1)
    h_i, h_c, h_o, dinv_b = _matmul_scale(x_pad, w_cat, deg_parts)

    zeros_init = jnp.zeros((N_PAD // NSUB, 128), jnp.float32)
    parts = _aggregate(h_i, h_c, h_o, src.reshape(E // WIN, WIN),
                       dst.reshape(E // WIN, WIN), zeros_init)
    parts = parts.reshape(NGATE, NSC, N_PAD, 128)

    b_i = (bi + bxi + bhi).reshape(1, 128)
    b_c = (bc + bxc + bhc).reshape(1, 128)
    b_o = (bo + bxo + bho).reshape(1, 128)
    h_out, c_out = _gates(parts, dinv_b, b_i, b_c, b_o, wco.reshape(1, 128))
    return h_out[:N], c_out[:N]


# speedup vs baseline: 40.0381x; 1.2111x over previous
import jax
import jax.numpy as jnp
from jax import lax
from jax.experimental import pallas as pl
from jax.experimental.pallas import tpu as pltpu
from jax.experimental.pallas import tpu_sc as plsc

N = 10000
N_PAD = 10240
E = 320000
D_IN = 128
D_OUT = 128
NGATE = 3
NSC = 2
NSUB = 16
ROWS16 = N_PAD // 16
EDGES_PER_WORKER = E // (NSC * NSUB)
EDGES_PER_CORE = E // NSC
EDGES_PER_SUB = EDGES_PER_CORE // NSUB
WIN = 100
NWIN = EDGES_PER_SUB // WIN


def _sc_mesh():
    return plsc.VectorSubcoreMesh(core_axis_name="c", subcore_axis_name="s")


_SC_PARAMS = pltpu.CompilerParams(needs_layout_passes=False,
                                  use_tc_tiling_on_sc=False)


def _degree_partials(dst):
    iota = lax.iota(jnp.int32, ROWS16)

    @pl.kernel(
        out_type=jax.ShapeDtypeStruct((NSC * ROWS16, 16), jnp.float32),
        mesh=_sc_mesh(),
        scratch_types=[
            pltpu.VMEM((EDGES_PER_WORKER,), jnp.int32),
            pltpu.VMEM((ROWS16, 16), jnp.float32),
            pltpu.VMEM((ROWS16,), jnp.int32),
            pltpu.VMEM_SHARED((ROWS16, 16), jnp.float32),
        ],
        compiler_params=_SC_PARAMS,
    )
    def deg_kernel(dst_hbm, iota_hbm, out_hbm, dst_v, hist_v, iota_v, deg_sh):
        cid = lax.axis_index("c")
        sid = lax.axis_index("s")
        wid = cid * NSUB + sid

        zeros16 = jnp.zeros((16,), jnp.float32)

        @pl.loop(0, ROWS16)
        def _(r):
            hist_v[r] = zeros16

        @pl.when(sid == 0)
        def _():
            pltpu.sync_copy(hist_v, deg_sh)

        plsc.subcore_barrier()

        pltpu.sync_copy(dst_hbm.at[pl.ds(wid * EDGES_PER_WORKER, EDGES_PER_WORKER)], dst_v)
        pltpu.sync_copy(iota_hbm, iota_v)

        ones16 = jnp.ones((16,), jnp.float32)

        @pl.loop(0, EDGES_PER_WORKER // 16)
        def _(i):
            v = dst_v[pl.ds(i * 16, 16)]
            row = jnp.right_shift(v, 4)
            lane = jnp.bitwise_and(v, 15)
            plsc.addupdate_scatter(hist_v, [row, lane], ones16)

        pltpu.sync_copy(hist_v, deg_sh.at[iota_v], add=True)
        plsc.subcore_barrier()

        rows_per_sub = ROWS16 // NSUB
        pltpu.sync_copy(
            deg_sh.at[pl.ds(sid * rows_per_sub, rows_per_sub)],
            out_hbm.at[pl.ds(cid * ROWS16 + sid * rows_per_sub, rows_per_sub)],
        )

    return deg_kernel(dst, iota)


def _matmul_scale(x_pad, w_cat, deg_parts):
    BLK = 2048
    grid = (N_PAD // BLK,)

    def body(x_ref, w_ref, d_ref, hi_ref, hc_ref, ho_ref, dv_ref):
        h = jnp.dot(x_ref[...], w_ref[...], preferred_element_type=jnp.float32)
        deg = d_ref[0] + d_ref[1]
        dinv = jnp.where(deg > 0.0, lax.rsqrt(jnp.maximum(deg, 1.0)), 0.0)
        hs = h * dinv
        hi_ref[...] = hs[:, :128]
        hc_ref[...] = hs[:, 128:256]
        ho_ref[...] = hs[:, 256:]
        dv_ref[...] = jnp.broadcast_to(dinv, (BLK, 128))

    out128 = jax.ShapeDtypeStruct((N_PAD, 128), jnp.float32)
    blk128 = pl.BlockSpec((BLK, 128), lambda i: (i, 0))
    return pl.pallas_call(
        body,
        grid=grid,
        in_specs=[
            pl.BlockSpec((BLK, D_IN), lambda i: (i, 0)),
            pl.BlockSpec((D_IN, NGATE * D_OUT), lambda i: (0, 0)),
            pl.BlockSpec((NSC, BLK, 1), lambda i: (0, i, 0)),
        ],
        out_specs=[blk128, blk128, blk128, blk128],
        out_shape=[out128, out128, out128, out128],
    )(x_pad, w_cat, deg_parts)


def _aggregate(h_i, h_c, h_o, src2d, dst2d, zeros_init):
    rows_per_sub = N_PAD // NSUB

    @pl.kernel(
        out_type=jax.ShapeDtypeStruct((NGATE * NSC * N_PAD, 128), jnp.float32),
        mesh=_sc_mesh(),
        scratch_types=[
            pltpu.VMEM((NWIN, WIN), jnp.int32),
            pltpu.VMEM((NWIN, WIN), jnp.int32),
            pltpu.VMEM((WIN, 128), jnp.float32),
            pltpu.VMEM((WIN, 128), jnp.float32),
            pltpu.VMEM_SHARED((N_PAD, 128), jnp.float32),
            pltpu.SemaphoreType.DMA,
            pltpu.SemaphoreType.DMA,
            pltpu.SemaphoreType.DMA,
            pltpu.SemaphoreType.DMA,
        ],
        compiler_params=_SC_PARAMS,
    )
    def agg_kernel(hi_hbm, hc_hbm, ho_hbm, src_hbm, dst_hbm, z_hbm, out_hbm,
                   src_v, dst_v, buf0, buf1, acc_sh, gsem0, gsem1, ssem0, ssem1):
        cid = lax.axis_index("c")
        sid = lax.axis_index("s")
        slc = pl.ds(sid * rows_per_sub, rows_per_sub)

        base_row = (cid * EDGES_PER_CORE + sid * EDGES_PER_SUB) // WIN
        pltpu.sync_copy(src_hbm.at[pl.ds(base_row, NWIN)], src_v)
        pltpu.sync_copy(dst_hbm.at[pl.ds(base_row, NWIN)], dst_v)

        for g, h_hbm in enumerate((hi_hbm, hc_hbm, ho_hbm)):
            pltpu.sync_copy(z_hbm, acc_sh.at[slc])
            pltpu.async_copy(h_hbm.at[src_v.at[0]], buf0, gsem0)
            pltpu.async_copy(h_hbm.at[src_v.at[1]], buf1, gsem1)
            plsc.subcore_barrier()

            @pl.loop(0, NWIN, step=2)
            def _(w):
                pltpu.make_async_copy(h_hbm.at[src_v.at[w]], buf0, gsem0).wait()
                sd0 = pltpu.async_copy(buf0, acc_sh.at[dst_v.at[w]], ssem0,
                                       add=True)
                sd0.wait()

                @pl.when(w + 2 < NWIN)
                def _():
                    pltpu.async_copy(h_hbm.at[src_v.at[w + 2]], buf0, gsem0)

                pltpu.make_async_copy(h_hbm.at[src_v.at[w + 1]], buf1,
                                      gsem1).wait()
                sd1 = pltpu.async_copy(buf1, acc_sh.at[dst_v.at[w + 1]], ssem1,
                                       add=True)
                sd1.wait()

                @pl.when(w + 3 < NWIN)
                def _():
                    pltpu.async_copy(h_hbm.at[src_v.at[w + 3]], buf1, gsem1)

            plsc.subcore_barrier()
            out_base = (g * NSC + cid) * N_PAD + sid * rows_per_sub
            pltpu.sync_copy(acc_sh.at[slc],
                            out_hbm.at[pl.ds(out_base, rows_per_sub)])
            plsc.subcore_barrier()

    return agg_kernel(h_i, h_c, h_o, src2d, dst2d, zeros_init)


def _gates(parts, dinv_b, b_i, b_c, b_o, wco):
    BLK = 2048
    grid = (N_PAD // BLK,)

    def body(p_ref, dv_ref, bi_ref, bc_ref, bo_ref, wco_ref, h_ref, c_ref):
        dv = dv_ref[...]
        acc_i = p_ref[0, 0] + p_ref[0, 1]
        acc_c = p_ref[1, 0] + p_ref[1, 1]
        acc_o = p_ref[2, 0] + p_ref[2, 1]
        i_gate = jax.nn.sigmoid(acc_i * dv + bi_ref[...])
        t_gate = jnp.tanh(acc_c * dv + bc_ref[...])
        c_new = i_gate * t_gate
        o_gate = jax.nn.sigmoid(acc_o * dv + wco_ref[...] * c_new + bo_ref[...])
        h_ref[...] = o_gate * jnp.tanh(c_new)
        c_ref[...] = c_new

    bias_spec = pl.BlockSpec((1, 128), lambda i: (0, 0))
    blk128 = pl.BlockSpec((BLK, 128), lambda i: (i, 0))
    return pl.pallas_call(
        body,
        grid=grid,
        in_specs=[
            pl.BlockSpec((NGATE, NSC, BLK, 128), lambda i: (0, 0, i, 0)),
            blk128,
            bias_spec, bias_spec, bias_spec, bias_spec,
        ],
        out_specs=[blk128, blk128],
        out_shape=[
            jax.ShapeDtypeStruct((N_PAD, 128), jnp.float32),
            jax.ShapeDtypeStruct((N_PAD, 128), jnp.float32),
        ],
    )(parts, dinv_b, b_i, b_c, b_o, wco)


def kernel(x, edge_index, Wxi, bxi, Whi, bhi, Wxf, bxf, Whf, bhf, Wxc, bxc,
           Whc, bhc, Wxo, bxo, Who, bho, wci, wcf, wco, bi, bf, bc, bo):
    src = edge_index[0].astype(jnp.int32)
    dst = edge_index[1].astype(jnp.int32)

    deg_parts = _degree_partials(dst)
    deg_parts = deg_parts.reshape(NSC, N_PAD, 1)

    x_pad = jnp.pad(x, ((0, N_PAD - N), (0, 0)))
    w_cat = jnp.concatenate([Wxi, Wxc, Wxo], axis=1)
    h_i, h_c, h_o, dinv_b = _matmul_scale(x_pad, w_cat, deg_parts)

    zeros_init = jnp.zeros((N_PAD // NSUB, 128), jnp.float32)
    parts = _aggregate(h_i, h_c, h_o, src.reshape(E // WIN, WIN),
                       dst.reshape(E // WIN, WIN), zeros_init)
    parts = parts.reshape(NGATE, NSC, N_PAD, 128)

    b_i = (bi + bxi + bhi).reshape(1, 128)
    b_c = (bc + bxc + bhc).reshape(1, 128)
    b_o = (bo + bxo + bho).reshape(1, 128)
    h_out, c_out = _gates(parts, dinv_b, b_i, b_c, b_o, wco.reshape(1, 128))
    return h_out[:N], c_out[:N]


# speedup vs baseline: 45.3112x; 1.1317x over previous
import jax
import jax.numpy as jnp
from jax import lax
from jax.experimental import pallas as pl
from jax.experimental.pallas import tpu as pltpu
from jax.experimental.pallas import tpu_sc as plsc

N = 10000
N_PAD = 10240
E = 320000
D_IN = 128
D_OUT = 128
NGATE = 3
NSC = 2
NSUB = 16
ROWS16 = N_PAD // 16
EDGES_PER_WORKER = E // (NSC * NSUB)
EDGES_PER_CORE = E // NSC
EDGES_PER_SUB = EDGES_PER_CORE // NSUB
WIN = 100
NWIN = EDGES_PER_SUB // WIN


def _sc_mesh():
    return plsc.VectorSubcoreMesh(core_axis_name="c", subcore_axis_name="s")


_SC_PARAMS = pltpu.CompilerParams(needs_layout_passes=False,
                                  use_tc_tiling_on_sc=False)


def _degree_partials(dst):
    iota = lax.iota(jnp.int32, ROWS16)

    @pl.kernel(
        out_type=jax.ShapeDtypeStruct((NSC * ROWS16, 16), jnp.float32),
        mesh=_sc_mesh(),
        scratch_types=[
            pltpu.VMEM((EDGES_PER_WORKER,), jnp.int32),
            pltpu.VMEM((ROWS16, 16), jnp.float32),
            pltpu.VMEM((ROWS16,), jnp.int32),
            pltpu.VMEM_SHARED((ROWS16, 16), jnp.float32),
        ],
        compiler_params=_SC_PARAMS,
    )
    def deg_kernel(dst_hbm, iota_hbm, out_hbm, dst_v, hist_v, iota_v, deg_sh):
        cid = lax.axis_index("c")
        sid = lax.axis_index("s")
        wid = cid * NSUB + sid

        zeros16 = jnp.zeros((16,), jnp.float32)

        @pl.loop(0, ROWS16)
        def _(r):
            hist_v[r] = zeros16

        @pl.when(sid == 0)
        def _():
            pltpu.sync_copy(hist_v, deg_sh)

        plsc.subcore_barrier()

        pltpu.sync_copy(dst_hbm.at[pl.ds(wid * EDGES_PER_WORKER, EDGES_PER_WORKER)], dst_v)
        pltpu.sync_copy(iota_hbm, iota_v)

        ones16 = jnp.ones((16,), jnp.float32)

        @pl.loop(0, EDGES_PER_WORKER // 16)
        def _(i):
            v = dst_v[pl.ds(i * 16, 16)]
            row = jnp.right_shift(v, 4)
            lane = jnp.bitwise_and(v, 15)
            plsc.addupdate_scatter(hist_v, [row, lane], ones16)

        pltpu.sync_copy(hist_v, deg_sh.at[iota_v], add=True)
        plsc.subcore_barrier()

        rows_per_sub = ROWS16 // NSUB
        pltpu.sync_copy(
            deg_sh.at[pl.ds(sid * rows_per_sub, rows_per_sub)],
            out_hbm.at[pl.ds(cid * ROWS16 + sid * rows_per_sub, rows_per_sub)],
        )

    return deg_kernel(dst, iota)


def _matmul_scale(x_pad, w_cat, deg_parts):
    BLK = 2048
    grid = (N_PAD // BLK,)

    def body(x_ref, w_ref, d_ref, hi_ref, hc_ref, ho_ref, dv_ref):
        h = jnp.dot(x_ref[...], w_ref[...], preferred_element_type=jnp.float32)
        deg = d_ref[0] + d_ref[1]
        dinv = jnp.where(deg > 0.0, lax.rsqrt(jnp.maximum(deg, 1.0)), 0.0)
        hs = h * dinv
        hi_ref[...] = hs[:, :128]
        hc_ref[...] = hs[:, 128:256]
        ho_ref[...] = hs[:, 256:]
        dv_ref[...] = jnp.broadcast_to(dinv, (BLK, 128))

    out128 = jax.ShapeDtypeStruct((N_PAD, 128), jnp.float32)
    blk128 = pl.BlockSpec((BLK, 128), lambda i: (i, 0))
    return pl.pallas_call(
        body,
        grid=grid,
        in_specs=[
            pl.BlockSpec((BLK, D_IN), lambda i: (i, 0)),
            pl.BlockSpec((D_IN, NGATE * D_OUT), lambda i: (0, 0)),
            pl.BlockSpec((NSC, BLK, 1), lambda i: (0, i, 0)),
        ],
        out_specs=[blk128, blk128, blk128, blk128],
        out_shape=[out128, out128, out128, out128],
    )(x_pad, w_cat, deg_parts)


def _aggregate(h_i, h_c, h_o, src2d, dst2d, zeros_init):
    rows_per_sub = N // NSUB
    RING = 3

    @pl.kernel(
        out_type=jax.ShapeDtypeStruct((NGATE * NSC * N_PAD, 128), jnp.float32),
        mesh=_sc_mesh(),
        scratch_types=[
            pltpu.VMEM((NWIN, WIN), jnp.int32),
            pltpu.VMEM((1, WIN), jnp.int32),
            pltpu.VMEM((1, WIN), jnp.int32),
            pltpu.VMEM((1, WIN), jnp.int32),
            pltpu.VMEM((WIN, 128), jnp.float32),
            pltpu.VMEM((WIN, 128), jnp.float32),
            pltpu.VMEM((WIN, 128), jnp.float32),
            pltpu.VMEM_SHARED((N, 128), jnp.float32),
        ] + [pltpu.SemaphoreType.DMA] * 9,
        compiler_params=_SC_PARAMS,
    )
    def agg_kernel(hi_hbm, hc_hbm, ho_hbm, src_hbm, dst_hbm, z_hbm, out_hbm,
                   src_v, db0, db1, db2, buf0, buf1, buf2, acc_sh,
                   gsem0, gsem1, gsem2, ssem0, ssem1, ssem2,
                   dsem0, dsem1, dsem2):
        cid = lax.axis_index("c")
        sid = lax.axis_index("s")
        slc = pl.ds(sid * rows_per_sub, rows_per_sub)
        dbufs = (db0, db1, db2)
        bufs = (buf0, buf1, buf2)
        gsems = (gsem0, gsem1, gsem2)
        ssems = (ssem0, ssem1, ssem2)
        dsems = (dsem0, dsem1, dsem2)

        base_row = (cid * EDGES_PER_CORE + sid * EDGES_PER_SUB) // WIN
        pltpu.sync_copy(src_hbm.at[pl.ds(base_row, NWIN)], src_v)

        NMAIN = NWIN - (NWIN % RING)

        def slot(w, k, h_hbm):
            pltpu.make_async_copy(h_hbm.at[src_v.at[w]], bufs[k],
                                  gsems[k]).wait()
            pltpu.make_async_copy(dst_hbm.at[pl.ds(base_row, 1)], dbufs[k],
                                  dsems[k]).wait()
            sd = pltpu.async_copy(bufs[k], acc_sh.at[dbufs[k].at[0]], ssems[k],
                                  add=True)
            sd.wait()

            @pl.when(w + RING < NWIN)
            def _():
                pltpu.async_copy(dst_hbm.at[pl.ds(base_row + w + RING, 1)],
                                 dbufs[k], dsems[k])
                pltpu.async_copy(h_hbm.at[src_v.at[w + RING]], bufs[k],
                                 gsems[k])

        for g, h_hbm in enumerate((hi_hbm, hc_hbm, ho_hbm)):
            pltpu.sync_copy(z_hbm.at[pl.ds(0, rows_per_sub)], acc_sh.at[slc])
            for k in range(RING):
                pltpu.async_copy(dst_hbm.at[pl.ds(base_row + k, 1)], dbufs[k],
                                 dsems[k])
                pltpu.async_copy(h_hbm.at[src_v.at[k]], bufs[k], gsems[k])
            plsc.subcore_barrier()

            @pl.loop(0, NMAIN, step=RING)
            def _(w):
                for k in range(RING):
                    slot(w + k, k, h_hbm)

            for k in range(NWIN % RING):
                slot(NMAIN + k, k, h_hbm)

            plsc.subcore_barrier()
            out_base = (g * NSC + cid) * N_PAD + sid * rows_per_sub
            pltpu.sync_copy(acc_sh.at[slc],
                            out_hbm.at[pl.ds(out_base, rows_per_sub)])
            plsc.subcore_barrier()

    return agg_kernel(h_i, h_c, h_o, src2d, dst2d, zeros_init)


def _gates(parts, dinv_b, b_i, b_c, b_o, wco):
    BLK = 2048
    grid = (N_PAD // BLK,)

    def body(p_ref, dv_ref, bi_ref, bc_ref, bo_ref, wco_ref, h_ref, c_ref):
        dv = dv_ref[...]
        acc_i = p_ref[0, 0] + p_ref[0, 1]
        acc_c = p_ref[1, 0] + p_ref[1, 1]
        acc_o = p_ref[2, 0] + p_ref[2, 1]
        i_gate = jax.nn.sigmoid(acc_i * dv + bi_ref[...])
        t_gate = jnp.tanh(acc_c * dv + bc_ref[...])
        c_new = i_gate * t_gate
        o_gate = jax.nn.sigmoid(acc_o * dv + wco_ref[...] * c_new + bo_ref[...])
        h_ref[...] = o_gate * jnp.tanh(c_new)
        c_ref[...] = c_new

    bias_spec = pl.BlockSpec((1, 128), lambda i: (0, 0))
    blk128 = pl.BlockSpec((BLK, 128), lambda i: (i, 0))
    return pl.pallas_call(
        body,
        grid=grid,
        in_specs=[
            pl.BlockSpec((NGATE, NSC, BLK, 128), lambda i: (0, 0, i, 0)),
            blk128,
            bias_spec, bias_spec, bias_spec, bias_spec,
        ],
        out_specs=[blk128, blk128],
        out_shape=[
            jax.ShapeDtypeStruct((N_PAD, 128), jnp.float32),
            jax.ShapeDtypeStruct((N_PAD, 128), jnp.float32),
        ],
    )(parts, dinv_b, b_i, b_c, b_o, wco)


def kernel(x, edge_index, Wxi, bxi, Whi, bhi, Wxf, bxf, Whf, bhf, Wxc, bxc,
           Whc, bhc, Wxo, bxo, Who, bho, wci, wcf, wco, bi, bf, bc, bo):
    src = edge_index[0].astype(jnp.int32)
    dst = edge_index[1].astype(jnp.int32)

    deg_parts = _degree_partials(dst)
    deg_parts = deg_parts.reshape(NSC, N_PAD, 1)

    x_pad = jnp.pad(x, ((0, N_PAD - N), (0, 0)))
    w_cat = jnp.concatenate([Wxi, Wxc, Wxo], axis=1)
    h_i, h_c, h_o, dinv_b = _matmul_scale(x_pad, w_cat, deg_parts)

    zeros_init = jnp.zeros((N_PAD // NSUB, 128), jnp.float32)
    parts = _aggregate(h_i, h_c, h_o, src.reshape(E // WIN, WIN),
                       dst.reshape(E // WIN, WIN), zeros_init)
    parts = parts.reshape(NGATE, NSC, N_PAD, 128)

    b_i = (bi + bxi + bhi).reshape(1, 128)
    b_c = (bc + bxc + bhc).reshape(1, 128)
    b_o = (bo + bxo + bho).reshape(1, 128)
    h_out, c_out = _gates(parts, dinv_b, b_i, b_c, b_o, wco.reshape(1, 128))
    return h_out[:N], c_out[:N]


# speedup vs baseline: 46.2641x; 1.0210x over previous
import jax
import jax.numpy as jnp
from jax import lax
from jax.experimental import pallas as pl
from jax.experimental.pallas import tpu as pltpu
from jax.experimental.pallas import tpu_sc as plsc

N = 10000
N_PAD = 10240
E = 320000
D_IN = 128
D_OUT = 128
NGATE = 3
NSC = 2
NSUB = 16
ROWS16 = N_PAD // 16
EDGES_PER_WORKER = E // (NSC * NSUB)
EDGES_PER_CORE = E // NSC
EDGES_PER_SUB = EDGES_PER_CORE // NSUB
WIN = 125
NWIN = EDGES_PER_SUB // WIN


def _sc_mesh():
    return plsc.VectorSubcoreMesh(core_axis_name="c", subcore_axis_name="s")


_SC_PARAMS = pltpu.CompilerParams(needs_layout_passes=False,
                                  use_tc_tiling_on_sc=False)


def _degree_partials(dst):
    iota = lax.iota(jnp.int32, ROWS16)

    @pl.kernel(
        out_type=jax.ShapeDtypeStruct((NSC * ROWS16, 16), jnp.float32),
        mesh=_sc_mesh(),
        scratch_types=[
            pltpu.VMEM((EDGES_PER_WORKER,), jnp.int32),
            pltpu.VMEM((ROWS16, 16), jnp.float32),
            pltpu.VMEM((ROWS16,), jnp.int32),
            pltpu.VMEM_SHARED((ROWS16, 16), jnp.float32),
        ],
        compiler_params=_SC_PARAMS,
    )
    def deg_kernel(dst_hbm, iota_hbm, out_hbm, dst_v, hist_v, iota_v, deg_sh):
        cid = lax.axis_index("c")
        sid = lax.axis_index("s")
        wid = cid * NSUB + sid

        zeros16 = jnp.zeros((16,), jnp.float32)

        @pl.loop(0, ROWS16)
        def _(r):
            hist_v[r] = zeros16

        @pl.when(sid == 0)
        def _():
            pltpu.sync_copy(hist_v, deg_sh)

        plsc.subcore_barrier()

        pltpu.sync_copy(dst_hbm.at[pl.ds(wid * EDGES_PER_WORKER, EDGES_PER_WORKER)], dst_v)
        pltpu.sync_copy(iota_hbm, iota_v)

        ones16 = jnp.ones((16,), jnp.float32)

        @pl.loop(0, EDGES_PER_WORKER // 16)
        def _(i):
            v = dst_v[pl.ds(i * 16, 16)]
            row = jnp.right_shift(v, 4)
            lane = jnp.bitwise_and(v, 15)
            plsc.addupdate_scatter(hist_v, [row, lane], ones16)

        pltpu.sync_copy(hist_v, deg_sh.at[iota_v], add=True)
        plsc.subcore_barrier()

        rows_per_sub = ROWS16 // NSUB
        pltpu.sync_copy(
            deg_sh.at[pl.ds(sid * rows_per_sub, rows_per_sub)],
            out_hbm.at[pl.ds(cid * ROWS16 + sid * rows_per_sub, rows_per_sub)],
        )

    return deg_kernel(dst, iota)


def _matmul_scale(x, w_cat, deg_parts):
    BLK = 2000
    grid = (N // BLK,)

    def body(x_ref, w_ref, d_ref, hi_ref, hc_ref, ho_ref):
        h = jnp.dot(x_ref[...], w_ref[...], preferred_element_type=jnp.float32)
        deg = d_ref[0] + d_ref[1]
        dinv = jnp.where(deg > 0.0, lax.rsqrt(jnp.maximum(deg, 1.0)), 0.0)
        hs = h * dinv
        hi_ref[...] = hs[:, :128]
        hc_ref[...] = hs[:, 128:256]
        ho_ref[...] = hs[:, 256:]

    out128 = jax.ShapeDtypeStruct((N, 128), jnp.float32)
    blk128 = pl.BlockSpec((BLK, 128), lambda i: (i, 0))
    return pl.pallas_call(
        body,
        grid=grid,
        in_specs=[
            pl.BlockSpec((BLK, D_IN), lambda i: (i, 0)),
            pl.BlockSpec((D_IN, NGATE * D_OUT), lambda i: (0, 0)),
            pl.BlockSpec((NSC, BLK, 1), lambda i: (0, i, 0)),
        ],
        out_specs=[blk128, blk128, blk128],
        out_shape=[out128, out128, out128],
    )(x, w_cat, deg_parts)


def _aggregate(h_i, h_c, h_o, src2d, dst2d, zeros_init):
    rows_per_sub = N // NSUB
    RING = 3
    IRING = 2 * RING

    @pl.kernel(
        out_type=jax.ShapeDtypeStruct((NGATE * NSC * N, 128), jnp.float32),
        mesh=_sc_mesh(),
        scratch_types=(
            [pltpu.VMEM((1, WIN), jnp.int32)] * IRING
            + [pltpu.VMEM((1, WIN), jnp.int32)] * IRING
            + [pltpu.VMEM((WIN, 128), jnp.float32)] * RING
            + [pltpu.VMEM_SHARED((N, 128), jnp.float32)]
            + [pltpu.SemaphoreType.DMA] * (3 * RING + 2 * IRING)
        ),
        compiler_params=_SC_PARAMS,
    )
    def agg_kernel(hi_hbm, hc_hbm, ho_hbm, src_hbm, dst_hbm, z_hbm, out_hbm,
                   *refs):
        sbufs = refs[:IRING]
        dbufs = refs[IRING:2 * IRING]
        bufs = refs[2 * IRING:2 * IRING + RING]
        acc_sh = refs[2 * IRING + RING]
        sems = refs[2 * IRING + RING + 1:]
        gsems = sems[:RING]
        ssems = sems[RING:2 * RING]
        rsems = sems[2 * RING:3 * RING]
        isems = sems[3 * RING:3 * RING + IRING]
        dsems = sems[3 * RING + IRING:]
        cid = lax.axis_index("c")
        sid = lax.axis_index("s")
        slc = pl.ds(sid * rows_per_sub, rows_per_sub)

        base_row = (cid * EDGES_PER_CORE + sid * EDGES_PER_SUB) // WIN
        NMAIN = NWIN - (NWIN % IRING)

        def load_idx(j, win_row):
            pltpu.async_copy(src_hbm.at[pl.ds(win_row, 1)], sbufs[j], isems[j])
            pltpu.async_copy(dst_hbm.at[pl.ds(win_row, 1)], dbufs[j], dsems[j])

        def wait_idx(j):
            pltpu.make_async_copy(src_hbm.at[pl.ds(base_row, 1)], sbufs[j],
                                  isems[j]).wait()
            pltpu.make_async_copy(dst_hbm.at[pl.ds(base_row, 1)], dbufs[j],
                                  dsems[j]).wait()

        def slot(w, j, h_hbm, refill):
            k = j % RING
            pltpu.make_async_copy(h_hbm.at[sbufs[j].at[0]], bufs[k],
                                  gsems[k]).wait()
            sd = pltpu.async_copy(bufs[k], acc_sh.at[dbufs[j].at[0]], ssems[k],
                                  add=True)
            sd.wait()
            if refill:
                @pl.when(w + IRING < NWIN)
                def _():
                    load_idx(j, base_row + w + IRING)

                @pl.when(w + RING < NWIN)
                def _():
                    j2 = (j + RING) % IRING
                    wait_idx(j2)
                    pltpu.async_copy(h_hbm.at[sbufs[j2].at[0]], bufs[k],
                                     gsems[k])

        for g, h_hbm in enumerate((hi_hbm, hc_hbm, ho_hbm)):
            pltpu.sync_copy(z_hbm, acc_sh.at[slc])
            for j in range(IRING):
                load_idx(j, base_row + j)
            for j in range(RING):
                wait_idx(j)
                pltpu.async_copy(h_hbm.at[sbufs[j].at[0]], bufs[j], gsems[j])
            plsc.subcore_barrier()

            @pl.loop(0, NMAIN, step=IRING)
            def _(w):
                for j in range(IRING):
                    slot(w + j, j, h_hbm, True)

            for j in range(NWIN % IRING):
                slot(NMAIN + j, j, h_hbm, False)

            plsc.subcore_barrier()
            out_base = (g * NSC + cid) * N + sid * rows_per_sub
            pltpu.sync_copy(acc_sh.at[slc],
                            out_hbm.at[pl.ds(out_base, rows_per_sub)])
            plsc.subcore_barrier()

    return agg_kernel(h_i, h_c, h_o, src2d, dst2d, zeros_init)


def _gates(parts, deg_parts, b_i, b_c, b_o, wco):
    BLK = 2000
    grid = (N // BLK,)

    def body(p_ref, d_ref, bi_ref, bc_ref, bo_ref, wco_ref, h_ref, c_ref):
        deg = d_ref[0] + d_ref[1]
        dv = jnp.where(deg > 0.0, lax.rsqrt(jnp.maximum(deg, 1.0)), 0.0)
        acc_i = p_ref[0, 0] + p_ref[0, 1]
        acc_c = p_ref[1, 0] + p_ref[1, 1]
        acc_o = p_ref[2, 0] + p_ref[2, 1]
        i_gate = jax.nn.sigmoid(acc_i * dv + bi_ref[...])
        t_gate = jnp.tanh(acc_c * dv + bc_ref[...])
        c_new = i_gate * t_gate
        o_gate = jax.nn.sigmoid(acc_o * dv + wco_ref[...] * c_new + bo_ref[...])
        h_ref[...] = o_gate * jnp.tanh(c_new)
        c_ref[...] = c_new

    bias_spec = pl.BlockSpec((1, 128), lambda i: (0, 0))
    blk128 = pl.BlockSpec((BLK, 128), lambda i: (i, 0))
    return pl.pallas_call(
        body,
        grid=grid,
        in_specs=[
            pl.BlockSpec((NGATE, NSC, BLK, 128), lambda i: (0, 0, i, 0)),
            pl.BlockSpec((NSC, BLK, 1), lambda i: (0, i, 0)),
            bias_spec, bias_spec, bias_spec, bias_spec,
        ],
        out_specs=[blk128, blk128],
        out_shape=[
            jax.ShapeDtypeStruct((N, 128), jnp.float32),
            jax.ShapeDtypeStruct((N, 128), jnp.float32),
        ],
    )(parts, deg_parts, b_i, b_c, b_o, wco)


def kernel(x, edge_index, Wxi, bxi, Whi, bhi, Wxf, bxf, Whf, bhf, Wxc, bxc,
           Whc, bhc, Wxo, bxo, Who, bho, wci, wcf, wco, bi, bf, bc, bo):
    src = edge_index[0].astype(jnp.int32)
    dst = edge_index[1].astype(jnp.int32)

    deg_parts = _degree_partials(dst)
    deg_parts = deg_parts.reshape(NSC, N_PAD, 1)

    w_cat = jnp.concatenate([Wxi, Wxc, Wxo], axis=1)
    h_i, h_c, h_o = _matmul_scale(x, w_cat, deg_parts)

    zeros_init = jnp.zeros((N // NSUB, 128), jnp.float32)
    parts = _aggregate(h_i, h_c, h_o, src.reshape(E // WIN, WIN),
                       dst.reshape(E // WIN, WIN), zeros_init)
    parts = parts.reshape(NGATE, NSC, N, 128)

    b_i = (bi + bxi + bhi).reshape(1, 128)
    b_c = (bc + bxc + bhc).reshape(1, 128)
    b_o = (bo + bxo + bho).reshape(1, 128)
    return _gates(parts, deg_parts, b_i, b_c, b_o, wco.reshape(1, 128))


# speedup vs baseline: 49.8314x; 1.0771x over previous
import jax
import jax.numpy as jnp
from jax import lax
from jax.experimental import pallas as pl
from jax.experimental.pallas import tpu as pltpu
from jax.experimental.pallas import tpu_sc as plsc

N = 10000
N_PAD = 10240
E = 320000
D_IN = 128
D_OUT = 128
NGATE = 3
NSC = 2
NSUB = 16
ROWS16 = N_PAD // 16
EDGES_PER_WORKER = E // (NSC * NSUB)
EDGES_PER_CORE = E // NSC
EDGES_PER_SUB = EDGES_PER_CORE // NSUB
WIN = 400
NWIN = EDGES_PER_SUB // WIN


def _sc_mesh():
    return plsc.VectorSubcoreMesh(core_axis_name="c", subcore_axis_name="s")


_SC_PARAMS = pltpu.CompilerParams(needs_layout_passes=False,
                                  use_tc_tiling_on_sc=False)


def _degree_partials(dst):
    iota = lax.iota(jnp.int32, ROWS16)

    @pl.kernel(
        out_type=jax.ShapeDtypeStruct((NSC * ROWS16, 16), jnp.float32),
        mesh=_sc_mesh(),
        scratch_types=[
            pltpu.VMEM((EDGES_PER_WORKER,), jnp.int32),
            pltpu.VMEM((ROWS16, 16), jnp.float32),
            pltpu.VMEM((ROWS16,), jnp.int32),
            pltpu.VMEM_SHARED((ROWS16, 16), jnp.float32),
        ],
        compiler_params=_SC_PARAMS,
    )
    def deg_kernel(dst_hbm, iota_hbm, out_hbm, dst_v, hist_v, iota_v, deg_sh):
        cid = lax.axis_index("c")
        sid = lax.axis_index("s")
        wid = cid * NSUB + sid

        zeros16 = jnp.zeros((16,), jnp.float32)

        @pl.loop(0, ROWS16)
        def _(r):
            hist_v[r] = zeros16

        @pl.when(sid == 0)
        def _():
            pltpu.sync_copy(hist_v, deg_sh)

        plsc.subcore_barrier()

        pltpu.sync_copy(dst_hbm.at[pl.ds(wid * EDGES_PER_WORKER, EDGES_PER_WORKER)], dst_v)
        pltpu.sync_copy(iota_hbm, iota_v)

        ones16 = jnp.ones((16,), jnp.float32)

        @pl.loop(0, EDGES_PER_WORKER // 16)
        def _(i):
            v = dst_v[pl.ds(i * 16, 16)]
            row = jnp.right_shift(v, 4)
            lane = jnp.bitwise_and(v, 15)
            plsc.addupdate_scatter(hist_v, [row, lane], ones16)

        pltpu.sync_copy(hist_v, deg_sh.at[iota_v], add=True)
        plsc.subcore_barrier()

        rows_per_sub = ROWS16 // NSUB
        pltpu.sync_copy(
            deg_sh.at[pl.ds(sid * rows_per_sub, rows_per_sub)],
            out_hbm.at[pl.ds(cid * ROWS16 + sid * rows_per_sub, rows_per_sub)],
        )

    return deg_kernel(dst, iota)


def _matmul_scale(x, w_cat, deg_parts):
    BLK = 2000
    grid = (N // BLK,)

    def body(x_ref, w_ref, d_ref, hi_ref, hc_ref, ho_ref):
        h = jnp.dot(x_ref[...], w_ref[...], preferred_element_type=jnp.float32)
        deg = d_ref[0] + d_ref[1]
        dinv = jnp.where(deg > 0.0, lax.rsqrt(jnp.maximum(deg, 1.0)), 0.0)
        hs = (h * dinv).astype(jnp.bfloat16)
        hi_ref[...] = hs[:, :128]
        hc_ref[...] = hs[:, 128:256]
        ho_ref[...] = hs[:, 256:]

    out128 = jax.ShapeDtypeStruct((N, 128), jnp.bfloat16)
    blk128 = pl.BlockSpec((BLK, 128), lambda i: (i, 0))
    return pl.pallas_call(
        body,
        grid=grid,
        in_specs=[
            pl.BlockSpec((BLK, D_IN), lambda i: (i, 0)),
            pl.BlockSpec((D_IN, NGATE * D_OUT), lambda i: (0, 0)),
            pl.BlockSpec((NSC, BLK, 1), lambda i: (0, i, 0)),
        ],
        out_specs=[blk128, blk128, blk128],
        out_shape=[out128, out128, out128],
    )(x, w_cat, deg_parts)


def _aggregate(h_i, h_c, h_o, src2d, dst2d, zeros_init):
    rows_per_sub = N // NSUB
    RING = 3
    IRING = 2 * RING

    @pl.kernel(
        out_type=jax.ShapeDtypeStruct((NGATE * NSC * N, 128), jnp.bfloat16),
        mesh=_sc_mesh(),
        scratch_types=(
            [pltpu.VMEM((1, WIN), jnp.int32)] * IRING
            + [pltpu.VMEM((1, WIN), jnp.int32)] * IRING
            + [pltpu.VMEM((WIN, 128), jnp.bfloat16)] * RING
            + [pltpu.VMEM_SHARED((N, 128), jnp.bfloat16)]
            + [pltpu.SemaphoreType.DMA] * (3 * RING + 2 * IRING)
        ),
        compiler_params=_SC_PARAMS,
    )
    def agg_kernel(hi_hbm, hc_hbm, ho_hbm, src_hbm, dst_hbm, z_hbm, out_hbm,
                   *refs):
        sbufs = refs[:IRING]
        dbufs = refs[IRING:2 * IRING]
        bufs = refs[2 * IRING:2 * IRING + RING]
        acc_sh = refs[2 * IRING + RING]
        sems = refs[2 * IRING + RING + 1:]
        gsems = sems[:RING]
        ssems = sems[RING:2 * RING]
        rsems = sems[2 * RING:3 * RING]
        isems = sems[3 * RING:3 * RING + IRING]
        dsems = sems[3 * RING + IRING:]
        cid = lax.axis_index("c")
        sid = lax.axis_index("s")
        slc = pl.ds(sid * rows_per_sub, rows_per_sub)

        base_row = (cid * EDGES_PER_CORE + sid * EDGES_PER_SUB) // WIN
        NMAIN = NWIN - (NWIN % IRING)

        def load_idx(j, win_row):
            pltpu.async_copy(src_hbm.at[pl.ds(win_row, 1)], sbufs[j], isems[j])
            pltpu.async_copy(dst_hbm.at[pl.ds(win_row, 1)], dbufs[j], dsems[j])

        def wait_idx(j):
            pltpu.make_async_copy(src_hbm.at[pl.ds(base_row, 1)], sbufs[j],
                                  isems[j]).wait()
            pltpu.make_async_copy(dst_hbm.at[pl.ds(base_row, 1)], dbufs[j],
                                  dsems[j]).wait()

        def slot(w, j, h_hbm, refill):
            k = j % RING
            pltpu.make_async_copy(h_hbm.at[sbufs[j].at[0]], bufs[k],
                                  gsems[k]).wait()
            sd = pltpu.async_copy(bufs[k], acc_sh.at[dbufs[j].at[0]], ssems[k],
                                  add=True)
            sd.wait()
            if refill:
                @pl.when(w + IRING < NWIN)
                def _():
                    load_idx(j, base_row + w + IRING)

                @pl.when(w + RING < NWIN)
                def _():
                    j2 = (j + RING) % IRING
                    wait_idx(j2)
                    pltpu.async_copy(h_hbm.at[sbufs[j2].at[0]], bufs[k],
                                     gsems[k])

        for g, h_hbm in enumerate((hi_hbm, hc_hbm, ho_hbm)):
            pltpu.sync_copy(z_hbm, acc_sh.at[slc])
            for j in range(IRING):
                load_idx(j, base_row + j)
            for j in range(RING):
                wait_idx(j)
                pltpu.async_copy(h_hbm.at[sbufs[j].at[0]], bufs[j], gsems[j])
            plsc.subcore_barrier()

            @pl.loop(0, NMAIN, step=IRING)
            def _(w):
                for j in range(IRING):
                    slot(w + j, j, h_hbm, True)

            for j in range(NWIN % IRING):
                slot(NMAIN + j, j, h_hbm, False)

            plsc.subcore_barrier()
            out_base = (g * NSC + cid) * N + sid * rows_per_sub
            pltpu.sync_copy(acc_sh.at[slc],
                            out_hbm.at[pl.ds(out_base, rows_per_sub)])
            plsc.subcore_barrier()

    return agg_kernel(h_i, h_c, h_o, src2d, dst2d, zeros_init)


def _gates(parts, deg_parts, b_i, b_c, b_o, wco):
    BLK = 2000
    grid = (N // BLK,)

    def body(p_ref, d_ref, bi_ref, bc_ref, bo_ref, wco_ref, h_ref, c_ref):
        deg = d_ref[0] + d_ref[1]
        dv = jnp.where(deg > 0.0, lax.rsqrt(jnp.maximum(deg, 1.0)), 0.0)
        p = p_ref[...].astype(jnp.float32)
        acc_i = p[0, 0] + p[0, 1]
        acc_c = p[1, 0] + p[1, 1]
        acc_o = p[2, 0] + p[2, 1]
        i_gate = jax.nn.sigmoid(acc_i * dv + bi_ref[...])
        t_gate = jnp.tanh(acc_c * dv + bc_ref[...])
        c_new = i_gate * t_gate
        o_gate = jax.nn.sigmoid(acc_o * dv + wco_ref[...] * c_new + bo_ref[...])
        h_ref[...] = o_gate * jnp.tanh(c_new)
        c_ref[...] = c_new

    bias_spec = pl.BlockSpec((1, 128), lambda i: (0, 0))
    blk128 = pl.BlockSpec((BLK, 128), lambda i: (i, 0))
    return pl.pallas_call(
        body,
        grid=grid,
        in_specs=[
            pl.BlockSpec((NGATE, NSC, BLK, 128), lambda i: (0, 0, i, 0)),
            pl.BlockSpec((NSC, BLK, 1), lambda i: (0, i, 0)),
            bias_spec, bias_spec, bias_spec, bias_spec,
        ],
        out_specs=[blk128, blk128],
        out_shape=[
            jax.ShapeDtypeStruct((N, 128), jnp.float32),
            jax.ShapeDtypeStruct((N, 128), jnp.float32),
        ],
    )(parts, deg_parts, b_i, b_c, b_o, wco)


def kernel(x, edge_index, Wxi, bxi, Whi, bhi, Wxf, bxf, Whf, bhf, Wxc, bxc,
           Whc, bhc, Wxo, bxo, Who, bho, wci, wcf, wco, bi, bf, bc, bo):
    src = edge_index[0].astype(jnp.int32)
    dst = edge_index[1].astype(jnp.int32)

    deg_parts = _degree_partials(dst)
    deg_parts = deg_parts.reshape(NSC, N_PAD, 1)

    w_cat = jnp.concatenate([Wxi, Wxc, Wxo], axis=1)
    h_i, h_c, h_o = _matmul_scale(x, w_cat, deg_parts)

    zeros_init = jnp.zeros((N // NSUB, 128), jnp.bfloat16)
    parts = _aggregate(h_i, h_c, h_o, src.reshape(E // WIN, WIN),
                       dst.reshape(E // WIN, WIN), zeros_init)
    parts = parts.reshape(NGATE, NSC, N, 128)

    b_i = (bi + bxi + bhi).reshape(1, 128)
    b_c = (bc + bxc + bhc).reshape(1, 128)
    b_o = (bo + bxo + bho).reshape(1, 128)
    return _gates(parts, deg_parts, b_i, b_c, b_o, wco.reshape(1, 128))


# speedup vs baseline: 50.9651x; 1.0227x over previous
import jax
import jax.numpy as jnp
from jax import lax
from jax.experimental import pallas as pl
from jax.experimental.pallas import tpu as pltpu
from jax.experimental.pallas import tpu_sc as plsc

N = 10000
N_PAD = 10240
E = 320000
D_IN = 128
D_OUT = 128
NGATE = 3
NSC = 2
NSUB = 16
ROWS16 = N_PAD // 16
EDGES_PER_WORKER = E // (NSC * NSUB)
EDGES_PER_CORE = E // NSC
EDGES_PER_SUB = EDGES_PER_CORE // NSUB
WIN = 250
NWIN = EDGES_PER_SUB // WIN


def _sc_mesh():
    return plsc.VectorSubcoreMesh(core_axis_name="c", subcore_axis_name="s")


_SC_PARAMS = pltpu.CompilerParams(needs_layout_passes=False,
                                  use_tc_tiling_on_sc=False)


def _degree_partials(dst):
    iota = lax.iota(jnp.int32, ROWS16)

    @pl.kernel(
        out_type=jax.ShapeDtypeStruct((NSC * ROWS16, 16), jnp.float32),
        mesh=_sc_mesh(),
        scratch_types=[
            pltpu.VMEM((EDGES_PER_WORKER,), jnp.int32),
            pltpu.VMEM((ROWS16, 16), jnp.float32),
            pltpu.VMEM((ROWS16,), jnp.int32),
            pltpu.VMEM_SHARED((ROWS16, 16), jnp.float32),
        ],
        compiler_params=_SC_PARAMS,
    )
    def deg_kernel(dst_hbm, iota_hbm, out_hbm, dst_v, hist_v, iota_v, deg_sh):
        cid = lax.axis_index("c")
        sid = lax.axis_index("s")
        wid = cid * NSUB + sid

        zeros16 = jnp.zeros((16,), jnp.float32)

        @pl.loop(0, ROWS16)
        def _(r):
            hist_v[r] = zeros16

        @pl.when(sid == 0)
        def _():
            pltpu.sync_copy(hist_v, deg_sh)

        plsc.subcore_barrier()

        pltpu.sync_copy(dst_hbm.at[pl.ds(wid * EDGES_PER_WORKER, EDGES_PER_WORKER)], dst_v)
        pltpu.sync_copy(iota_hbm, iota_v)

        ones16 = jnp.ones((16,), jnp.float32)

        @pl.loop(0, EDGES_PER_WORKER // 16)
        def _(i):
            v = dst_v[pl.ds(i * 16, 16)]
            row = jnp.right_shift(v, 4)
            lane = jnp.bitwise_and(v, 15)
            plsc.addupdate_scatter(hist_v, [row, lane], ones16)

        pltpu.sync_copy(hist_v, deg_sh.at[iota_v], add=True)
        plsc.subcore_barrier()

        rows_per_sub = ROWS16 // NSUB
        pltpu.sync_copy(
            deg_sh.at[pl.ds(sid * rows_per_sub, rows_per_sub)],
            out_hbm.at[pl.ds(cid * ROWS16 + sid * rows_per_sub, rows_per_sub)],
        )

    return deg_kernel(dst, iota)


def _matmul_scale(x, w_cat, deg_parts):
    BLK = 2000
    grid = (N // BLK,)

    def body(x_ref, w_ref, d_ref, hi_ref, hc_ref, ho_ref):
        h = jnp.dot(x_ref[...], w_ref[...], preferred_element_type=jnp.float32)
        deg = d_ref[0] + d_ref[1]
        dinv = jnp.where(deg > 0.0, lax.rsqrt(jnp.maximum(deg, 1.0)), 0.0)
        hs = (h * dinv).astype(jnp.bfloat16)
        hi_ref[...] = hs[:, :128]
        hc_ref[...] = hs[:, 128:256]
        ho_ref[...] = hs[:, 256:]

    out128 = jax.ShapeDtypeStruct((N, 128), jnp.bfloat16)
    blk128 = pl.BlockSpec((BLK, 128), lambda i: (i, 0))
    return pl.pallas_call(
        body,
        grid=grid,
        in_specs=[
            pl.BlockSpec((BLK, D_IN), lambda i: (i, 0)),
            pl.BlockSpec((D_IN, NGATE * D_OUT), lambda i: (0, 0)),
            pl.BlockSpec((NSC, BLK, 1), lambda i: (0, i, 0)),
        ],
        out_specs=[blk128, blk128, blk128],
        out_shape=[out128, out128, out128],
    )(x, w_cat, deg_parts)


def _aggregate(h_i, h_c, h_o, src2d, dst2d, zeros_init):
    rows_per_sub = N // NSUB
    RING = 4
    IRING = 2 * RING

    @pl.kernel(
        out_type=jax.ShapeDtypeStruct((NGATE * NSC * N, 128), jnp.bfloat16),
        mesh=_sc_mesh(),
        scratch_types=(
            [pltpu.VMEM((1, WIN), jnp.int32)] * IRING
            + [pltpu.VMEM((1, WIN), jnp.int32)] * IRING
            + [pltpu.VMEM((WIN, 128), jnp.bfloat16)] * RING
            + [pltpu.VMEM_SHARED((N, 128), jnp.bfloat16)]
            + [pltpu.SemaphoreType.DMA] * (3 * RING + 2 * IRING)
        ),
        compiler_params=_SC_PARAMS,
    )
    def agg_kernel(hi_hbm, hc_hbm, ho_hbm, src_hbm, dst_hbm, z_hbm, out_hbm,
                   *refs):
        sbufs = refs[:IRING]
        dbufs = refs[IRING:2 * IRING]
        bufs = refs[2 * IRING:2 * IRING + RING]
        acc_sh = refs[2 * IRING + RING]
        sems = refs[2 * IRING + RING + 1:]
        gsems = sems[:RING]
        ssems = sems[RING:2 * RING]
        rsems = sems[2 * RING:3 * RING]
        isems = sems[3 * RING:3 * RING + IRING]
        dsems = sems[3 * RING + IRING:]
        cid = lax.axis_index("c")
        sid = lax.axis_index("s")
        slc = pl.ds(sid * rows_per_sub, rows_per_sub)

        base_row = (cid * EDGES_PER_CORE + sid * EDGES_PER_SUB) // WIN
        NMAIN = NWIN - (NWIN % IRING)

        def load_idx(j, win_row):
            pltpu.async_copy(src_hbm.at[pl.ds(win_row, 1)], sbufs[j], isems[j])
            pltpu.async_copy(dst_hbm.at[pl.ds(win_row, 1)], dbufs[j], dsems[j])

        def wait_idx(j):
            pltpu.make_async_copy(src_hbm.at[pl.ds(base_row, 1)], sbufs[j],
                                  isems[j]).wait()
            pltpu.make_async_copy(dst_hbm.at[pl.ds(base_row, 1)], dbufs[j],
                                  dsems[j]).wait()

        def slot(w, j, h_hbm, refill):
            k = j % RING
            pltpu.make_async_copy(h_hbm.at[sbufs[j].at[0]], bufs[k],
                                  gsems[k]).wait()
            sd = pltpu.async_copy(bufs[k], acc_sh.at[dbufs[j].at[0]], ssems[k],
                                  add=True)
            sd.wait()
            if refill:
                @pl.when(w + IRING < NWIN)
                def _():
                    load_idx(j, base_row + w + IRING)

                @pl.when(w + RING < NWIN)
                def _():
                    j2 = (j + RING) % IRING
                    wait_idx(j2)
                    pltpu.async_copy(h_hbm.at[sbufs[j2].at[0]], bufs[k],
                                     gsems[k])

        for g, h_hbm in enumerate((hi_hbm, hc_hbm, ho_hbm)):
            pltpu.sync_copy(z_hbm, acc_sh.at[slc])
            for j in range(IRING):
                load_idx(j, base_row + j)
            for j in range(RING):
                wait_idx(j)
                pltpu.async_copy(h_hbm.at[sbufs[j].at[0]], bufs[j], gsems[j])
            plsc.subcore_barrier()

            @pl.loop(0, NMAIN, step=IRING)
            def _(w):
                for j in range(IRING):
                    slot(w + j, j, h_hbm, True)

            for j in range(NWIN % IRING):
                slot(NMAIN + j, j, h_hbm, False)

            plsc.subcore_barrier()
            out_base = (g * NSC + cid) * N + sid * rows_per_sub
            pltpu.sync_copy(acc_sh.at[slc],
                            out_hbm.at[pl.ds(out_base, rows_per_sub)])
            plsc.subcore_barrier()

    return agg_kernel(h_i, h_c, h_o, src2d, dst2d, zeros_init)


def _gates(parts, deg_parts, b_i, b_c, b_o, wco):
    BLK = 2000
    grid = (N // BLK,)

    def body(p_ref, d_ref, bi_ref, bc_ref, bo_ref, wco_ref, h_ref, c_ref):
        deg = d_ref[0] + d_ref[1]
        dv = jnp.where(deg > 0.0, lax.rsqrt(jnp.maximum(deg, 1.0)), 0.0)
        p = p_ref[...].astype(jnp.float32)
        acc_i = p[0, 0] + p[0, 1]
        acc_c = p[1, 0] + p[1, 1]
        acc_o = p[2, 0] + p[2, 1]
        i_gate = jax.nn.sigmoid(acc_i * dv + bi_ref[...])
        t_gate = jnp.tanh(acc_c * dv + bc_ref[...])
        c_new = i_gate * t_gate
        o_gate = jax.nn.sigmoid(acc_o * dv + wco_ref[...] * c_new + bo_ref[...])
        h_ref[...] = o_gate * jnp.tanh(c_new)
        c_ref[...] = c_new

    bias_spec = pl.BlockSpec((1, 128), lambda i: (0, 0))
    blk128 = pl.BlockSpec((BLK, 128), lambda i: (i, 0))
    return pl.pallas_call(
        body,
        grid=grid,
        in_specs=[
            pl.BlockSpec((NGATE, NSC, BLK, 128), lambda i: (0, 0, i, 0)),
            pl.BlockSpec((NSC, BLK, 1), lambda i: (0, i, 0)),
            bias_spec, bias_spec, bias_spec, bias_spec,
        ],
        out_specs=[blk128, blk128],
        out_shape=[
            jax.ShapeDtypeStruct((N, 128), jnp.float32),
            jax.ShapeDtypeStruct((N, 128), jnp.float32),
        ],
    )(parts, deg_parts, b_i, b_c, b_o, wco)


def kernel(x, edge_index, Wxi, bxi, Whi, bhi, Wxf, bxf, Whf, bhf, Wxc, bxc,
           Whc, bhc, Wxo, bxo, Who, bho, wci, wcf, wco, bi, bf, bc, bo):
    src = edge_index[0].astype(jnp.int32)
    dst = edge_index[1].astype(jnp.int32)

    deg_parts = _degree_partials(dst)
    deg_parts = deg_parts.reshape(NSC, N_PAD, 1)

    w_cat = jnp.concatenate([Wxi, Wxc, Wxo], axis=1)
    h_i, h_c, h_o = _matmul_scale(x, w_cat, deg_parts)

    zeros_init = jnp.zeros((N // NSUB, 128), jnp.bfloat16)
    parts = _aggregate(h_i, h_c, h_o, src.reshape(E // WIN, WIN),
                       dst.reshape(E // WIN, WIN), zeros_init)
    parts = parts.reshape(NGATE, NSC, N, 128)

    b_i = (bi + bxi + bhi).reshape(1, 128)
    b_c = (bc + bxc + bhc).reshape(1, 128)
    b_o = (bo + bxo + bho).reshape(1, 128)
    return _gates(parts, deg_parts, b_i, b_c, b_o, wco.reshape(1, 128))
